# Initial kernel scaffold; baseline (speedup 1.0000x reference)
import jax
import jax.numpy as jnp
from jax.experimental import pallas as pl


def kernel(logits, label_ids):
    raise NotImplementedError("write your pallas kernel here")


# speedup vs baseline: 28.8130x; 28.8130x over previous
import functools

import jax
import jax.numpy as jnp
from jax.experimental import pallas as pl
from jax.experimental.pallas import tpu as pltpu

E = 0.1
CENTER = 1.0 - E
B, S, L = 16, 2048, 512
N = B * S
ROWS = 512
NBLK = N // ROWS


def _loss_block(logits_ref, slab_ref, out_ref):
    i = pl.program_id(0)
    x = logits_ref[0]
    labs = slab_ref[0]

    c = (jax.lax.broadcasted_iota(jnp.int32, (ROWS, 1), 0)
         + (i % (S // ROWS)) * ROWS)

    iota_l = jax.lax.broadcasted_iota(jnp.int32, (ROWS, L), 1)

    def edge_val(cp):
        v = jnp.full(cp.shape, E / 4, jnp.float32)
        v = jnp.where((cp == 1) | (cp == S - 2), E / 3, v)
        v = jnp.where((cp == 0) | (cp == S - 1), E / 2, v)
        return v

    r = []
    for j in range(-2, 3):
        rj = labs[:, j + 2][:, None]
        valid = (c + j >= 0) & (c + j < S)
        r.append(jnp.where(valid, rj, -1))

    dot = jnp.zeros((ROWS, 1), jnp.float32)
    wsum = jnp.zeros((ROWS, 1), jnp.float32)
    for j in range(-2, 3):
        rj = r[j + 2]
        bound = (rj >= 0) & (rj < 16) & (rj % 2 == 1)
        keep = bound
        for jp in range(j + 1, 3):
            keep = keep & (r[jp + 2] != rj)
        w = jnp.where(keep,
                      jnp.float32(CENTER) if j == 0 else edge_val(c + j),
                      0.0)
        g = jnp.sum(jnp.where(iota_l == rj, x, 0.0), axis=1, keepdims=True)
        dot = dot + w * g
        wsum = wsum + w

    r0 = r[2]
    bound0 = (r0 < 16) & (r0 % 2 == 1)
    g0 = jnp.sum(jnp.where(iota_l == r0, x, 0.0), axis=1, keepdims=True)
    dot = dot + jnp.where(bound0, 0.0, g0)
    wsum = wsum + jnp.where(bound0, 0.0, 1.0)

    m = jnp.max(x, axis=1, keepdims=True)
    lse = m + jnp.log(jnp.sum(jnp.exp(x - m), axis=1, keepdims=True))

    part = jnp.sum(wsum * lse - dot)

    @pl.when(i == 0)
    def _():
        out_ref[0, 0] = 0.0
    out_ref[0, 0] += part


@jax.jit
def kernel(logits, label_ids):
    labpad = jnp.pad(label_ids, (2, 2), constant_values=-1)
    slab = jnp.stack([labpad[j:j + N] for j in range(5)]
                     + [jnp.full((N,), -1, jnp.int32)] * 3, axis=1)
    slab = slab.reshape(NBLK, ROWS, 8)
    logits3 = logits.reshape(NBLK, ROWS, L)

    out = pl.pallas_call(
        _loss_block,
        grid=(NBLK,),
        in_specs=[
            pl.BlockSpec((1, ROWS, L), lambda i: (i, 0, 0)),
            pl.BlockSpec((1, ROWS, 8), lambda i: (i, 0, 0)),
        ],
        out_specs=pl.BlockSpec(memory_space=pltpu.SMEM),
        out_shape=jax.ShapeDtypeStruct((1, 1), jnp.float32),
    )(logits3, slab)
    return (out[0, 0] / N).astype(jnp.float32)


# speedup vs baseline: 30.5247x; 1.0594x over previous
import jax
import jax.numpy as jnp
from jax.experimental import pallas as pl
from jax.experimental.pallas import tpu as pltpu

E = 0.1
CENTER = 1.0 - E
B, S, L = 16, 2048, 512
N = B * S
ROWS = 512
NBLK = N // ROWS


def _loss_block(logits_ref, slab_ref, out_ref):
    i = pl.program_id(0)
    x = logits_ref[0]
    labs = slab_ref[0]

    c = (jax.lax.broadcasted_iota(jnp.int32, (ROWS, 1), 0)
         + (i % (S // ROWS)) * ROWS)

    def edge_val(cp):
        v = jnp.full(cp.shape, E / 4, jnp.float32)
        v = jnp.where((cp == 1) | (cp == S - 2), E / 3, v)
        v = jnp.where((cp == 0) | (cp == S - 1), E / 2, v)
        return v

    r = []
    for j in range(-2, 3):
        rj = labs[:, j + 2][:, None]
        valid = (c + j >= 0) & (c + j < S)
        r.append(jnp.where(valid, rj, -1))

    xb = x[:, :16]
    iota_b = jax.lax.broadcasted_iota(jnp.int32, (ROWS, 16), 1)
    w16 = jnp.zeros((ROWS, 16), jnp.float32)
    wsum = jnp.zeros((ROWS, 1), jnp.float32)
    for j in range(-2, 3):
        rj = r[j + 2]
        bound = (rj >= 0) & (rj < 16) & (rj % 2 == 1)
        keep = bound
        for jp in range(j + 1, 3):
            keep = keep & (r[jp + 2] != rj)
        w = jnp.where(keep,
                      jnp.float32(CENTER) if j == 0 else edge_val(c + j),
                      0.0)
        w16 = w16 + jnp.where(iota_b == rj, w, 0.0)
        wsum = wsum + w
    dot = jnp.sum(w16 * xb, axis=1, keepdims=True)

    r0 = r[2]
    bound0 = (r0 < 16) & (r0 % 2 == 1)
    iota_l = jax.lax.broadcasted_iota(jnp.int32, (ROWS, L), 1)
    g0 = jnp.sum(jnp.where(iota_l == r0, x, 0.0), axis=1, keepdims=True)
    dot = dot + jnp.where(bound0, 0.0, g0)
    wsum = wsum + jnp.where(bound0, 0.0, 1.0)

    m = jnp.max(x, axis=1, keepdims=True)
    lse = m + jnp.log(jnp.sum(jnp.exp(x - m), axis=1, keepdims=True))

    out_ref[0, 0, 0] = jnp.sum(wsum * lse - dot)


@jax.jit
def kernel(logits, label_ids):
    labpad = jnp.pad(label_ids, (2, 2), constant_values=-1)
    slab = jnp.stack([labpad[j:j + N] for j in range(5)]
                     + [jnp.full((N,), -1, jnp.int32)] * 3, axis=1)
    slab = slab.reshape(NBLK, ROWS, 8)
    logits3 = logits.reshape(NBLK, ROWS, L)

    parts = pl.pallas_call(
        _loss_block,
        grid=(NBLK,),
        in_specs=[
            pl.BlockSpec((1, ROWS, L), lambda i: (i, 0, 0)),
            pl.BlockSpec((1, ROWS, 8), lambda i: (i, 0, 0)),
        ],
        out_specs=pl.BlockSpec((1, 1, 1), lambda i: (i, 0, 0),
                               memory_space=pltpu.SMEM),
        out_shape=jax.ShapeDtypeStruct((NBLK, 1, 1), jnp.float32),
        compiler_params=pltpu.CompilerParams(
            dimension_semantics=("parallel",)),
    )(logits3, slab)
    return (jnp.sum(parts) / N).astype(jnp.float32)


# speedup vs baseline: 71.4042x; 2.3392x over previous
import jax
import jax.numpy as jnp
from jax import lax
from jax.experimental import pallas as pl
from jax.experimental.pallas import tpu as pltpu
from jax.experimental.pallas import tpu_sc as plsc

E = 0.1
CENTER = 1.0 - E
B, S, L = 16, 2048, 512
N = B * S
NC, NS = 2, 16
NW = NC * NS
TPW = N // NW
NGRP = TPW // 16
ROWS = 512
NBLK = N // ROWS


def _sc_body(labpad_hbm, flat_hbm, dot_hbm, wsum_hbm,
             labs_v, idx_v, w_v, val_v, dot_v, wsum_v, sem):
    cid = lax.axis_index("c")
    sid = lax.axis_index("s")
    wid = sid * NC + cid
    base = wid * TPW
    pltpu.sync_copy(labpad_hbm.at[pl.ds(base, TPW + 16)], labs_v)
    cbase = (wid % (S // TPW)) * TPW

    def stencil(g, carry):
        t0 = g * 16
        lane = lax.iota(jnp.int32, 16)
        tloc = t0 + lane
        c = cbase + tloc
        rowbase = (base + tloc) * L
        rs = []
        for j in range(-2, 3):
            rj = labs_v[pl.ds(t0 + 8 + j, 16)]
            vj = ((c + j) >= 0) & ((c + j) < S)
            rs.append(jnp.where(vj, rj, -1))
        for j in range(-2, 3):
            rj = rs[j + 2]
            bnd = (rj >= 0) & (rj < 16) & ((rj & 1) == 1)
            keep = bnd
            for jp in range(j + 1, 3):
                keep = keep & (rs[jp + 2] != rj)
            if j == 0:
                val = jnp.full(16, CENTER, jnp.float32)
            else:
                cp = c + j
                val = jnp.full(16, E / 4, jnp.float32)
                val = jnp.where((cp == 1) | (cp == S - 2), E / 3, val)
                val = jnp.where((cp == 0) | (cp == S - 1), E / 2, val)
            sl = (j + 2) * TPW + t0
            w_v[pl.ds(sl, 16)] = jnp.where(keep, val, 0.0)
            idx_v[pl.ds(sl, 16)] = rowbase + jnp.where(keep, rj, 0)
        r0 = rs[2]
        bnd0 = (r0 < 16) & ((r0 & 1) == 1)
        idx_v[pl.ds(5 * TPW + t0, 16)] = rowbase + r0
        w_v[pl.ds(5 * TPW + t0, 16)] = jnp.where(bnd0, 0.0, 1.0)
        return carry

    lax.fori_loop(0, NGRP, stencil, 0)

    copies = []
    for k in range(6 * TPW // 128):
        copies.append(pltpu.async_copy(
            flat_hbm.at[idx_v.at[pl.ds(k * 128, 128)]],
            val_v.at[pl.ds(k * 128, 128)], sem))
    for cp in copies:
        cp.wait()

    def combine(g, carry):
        t0 = g * 16
        dot = jnp.zeros(16, jnp.float32)
        ws = jnp.zeros(16, jnp.float32)
        for sl in range(6):
            w = w_v[pl.ds(sl * TPW + t0, 16)]
            dot = dot + w * val_v[pl.ds(sl * TPW + t0, 16)]
            ws = ws + w
        dot_v[pl.ds(t0, 16)] = dot
        wsum_v[pl.ds(t0, 16)] = ws
        return carry

    lax.fori_loop(0, NGRP, combine, 0)
    pltpu.sync_copy(dot_v, dot_hbm.at[pl.ds(base, TPW)])
    pltpu.sync_copy(wsum_v, wsum_hbm.at[pl.ds(base, TPW)])


def _sc_sparse_part(logits, label_ids):
    labpad = jnp.pad(label_ids, (8, 8), constant_values=-1)
    flat = logits.reshape(N * L)
    mesh = plsc.VectorSubcoreMesh(core_axis_name="c", subcore_axis_name="s",
                                  num_cores=NC, num_subcores=NS)
    k = pl.kernel(
        _sc_body,
        out_type=(jax.ShapeDtypeStruct((N,), jnp.float32),
                  jax.ShapeDtypeStruct((N,), jnp.float32)),
        mesh=mesh,
        scratch_types=[
            pltpu.VMEM((TPW + 16,), jnp.int32),
            pltpu.VMEM((6 * TPW,), jnp.int32),
            pltpu.VMEM((6 * TPW,), jnp.float32),
            pltpu.VMEM((6 * TPW,), jnp.float32),
            pltpu.VMEM((TPW,), jnp.float32),
            pltpu.VMEM((TPW,), jnp.float32),
            pltpu.SemaphoreType.DMA,
        ],
        compiler_params=pltpu.CompilerParams(use_tc_tiling_on_sc=False,
                                             needs_layout_passes=False),
    )
    return k(labpad, flat)


def _tc_body(x_ref, dot_ref, wsum_ref, out_ref):
    x = x_ref[0]
    dot = dot_ref[0]
    ws = wsum_ref[0]
    m = jnp.max(x, axis=1, keepdims=True)
    lse = m + jnp.log(jnp.sum(jnp.exp(x - m), axis=1, keepdims=True))
    out_ref[0, 0, 0] = jnp.sum(ws * lse - dot)


@jax.jit
def kernel(logits, label_ids):
    dot, wsum = _sc_sparse_part(logits, label_ids)
    parts = pl.pallas_call(
        _tc_body,
        grid=(NBLK,),
        in_specs=[
            pl.BlockSpec((1, ROWS, L), lambda i: (i, 0, 0)),
            pl.BlockSpec((1, ROWS, 1), lambda i: (i, 0, 0)),
            pl.BlockSpec((1, ROWS, 1), lambda i: (i, 0, 0)),
        ],
        out_specs=pl.BlockSpec((1, 1, 1), lambda i: (i, 0, 0),
                               memory_space=pltpu.SMEM),
        out_shape=jax.ShapeDtypeStruct((NBLK, 1, 1), jnp.float32),
        compiler_params=pltpu.CompilerParams(
            dimension_semantics=("parallel",)),
    )(logits.reshape(NBLK, ROWS, L),
      dot.reshape(NBLK, ROWS, 1),
      wsum.reshape(NBLK, ROWS, 1))
    return (jnp.sum(parts) / N).astype(jnp.float32)


# speedup vs baseline: 71.5073x; 1.0014x over previous
import jax
import jax.numpy as jnp
from jax import lax
from jax.experimental import pallas as pl
from jax.experimental.pallas import tpu as pltpu
from jax.experimental.pallas import tpu_sc as plsc

E = 0.1
CENTER = 1.0 - E
B, S, L = 16, 2048, 512
N = B * S
NC, NS = 2, 16
NW = NC * NS
TPW = N // NW
NGRP = TPW // 16
ROWS = 512
NBLK = N // ROWS


def _sc_body(labpad_hbm, flat_hbm, dot_hbm, wsum_hbm,
             labs_v, idx_v, w_v, val_v, dot_v, wsum_v, sem):
    cid = lax.axis_index("c")
    sid = lax.axis_index("s")
    wid = sid * NC + cid
    base = wid * TPW
    pltpu.sync_copy(labpad_hbm.at[pl.ds(base, TPW + 16)], labs_v)
    cbase = (wid % (S // TPW)) * TPW

    def stencil(g, carry):
        t0 = g * 16
        lane = lax.iota(jnp.int32, 16)
        tloc = t0 + lane
        c = cbase + tloc
        rowbase = (base + tloc) * L
        rs = []
        for j in range(-2, 3):
            rj = labs_v[pl.ds(t0 + 8 + j, 16)]
            vj = ((c + j) >= 0) & ((c + j) < S)
            rs.append(jnp.where(vj, rj, -1))
        for j in range(-2, 3):
            rj = rs[j + 2]
            bnd = (rj >= 0) & (rj < 16) & ((rj & 1) == 1)
            keep = bnd
            for jp in range(j + 1, 3):
                keep = keep & (rs[jp + 2] != rj)
            if j == 0:
                val = jnp.full(16, CENTER, jnp.float32)
            else:
                cp = c + j
                val = jnp.full(16, E / 4, jnp.float32)
                val = jnp.where((cp == 1) | (cp == S - 2), E / 3, val)
                val = jnp.where((cp == 0) | (cp == S - 1), E / 2, val)
            sl = (j + 2) * TPW + t0
            w_v[pl.ds(sl, 16)] = jnp.where(keep, val, 0.0)
            idx_v[pl.ds(sl, 16)] = rowbase + jnp.where(keep, rj, 0)
        r0 = rs[2]
        bnd0 = (r0 < 16) & ((r0 & 1) == 1)
        idx_v[pl.ds(5 * TPW + t0, 16)] = rowbase + r0
        w_v[pl.ds(5 * TPW + t0, 16)] = jnp.where(bnd0, 0.0, 1.0)
        return carry

    lax.fori_loop(0, NGRP, stencil, 0)

    copies = []
    for k in range(6 * TPW // 128):
        copies.append(pltpu.async_copy(
            flat_hbm.at[idx_v.at[pl.ds(k * 128, 128)]],
            val_v.at[pl.ds(k * 128, 128)], sem))
    for cp in copies:
        cp.wait()

    def combine(g, carry):
        t0 = g * 16
        dot = jnp.zeros(16, jnp.float32)
        ws = jnp.zeros(16, jnp.float32)
        for sl in range(6):
            w = w_v[pl.ds(sl * TPW + t0, 16)]
            dot = dot + w * val_v[pl.ds(sl * TPW + t0, 16)]
            ws = ws + w
        dot_v[pl.ds(t0, 16)] = dot
        wsum_v[pl.ds(t0, 16)] = ws
        return carry

    lax.fori_loop(0, NGRP, combine, 0)
    pltpu.sync_copy(dot_v, dot_hbm.at[pl.ds(base, TPW)])
    pltpu.sync_copy(wsum_v, wsum_hbm.at[pl.ds(base, TPW)])


def _sc_sparse_part(logits, label_ids):
    labpad = jnp.pad(label_ids, (8, 8), constant_values=-1)
    flat = logits.reshape(N * L)
    mesh = plsc.VectorSubcoreMesh(core_axis_name="c", subcore_axis_name="s",
                                  num_cores=NC, num_subcores=NS)
    k = pl.kernel(
        _sc_body,
        out_type=(jax.ShapeDtypeStruct((N,), jnp.float32),
                  jax.ShapeDtypeStruct((N,), jnp.float32)),
        mesh=mesh,
        scratch_types=[
            pltpu.VMEM((TPW + 16,), jnp.int32),
            pltpu.VMEM((6 * TPW,), jnp.int32),
            pltpu.VMEM((6 * TPW,), jnp.float32),
            pltpu.VMEM((6 * TPW,), jnp.float32),
            pltpu.VMEM((TPW,), jnp.float32),
            pltpu.VMEM((TPW,), jnp.float32),
            pltpu.SemaphoreType.DMA,
        ],
    )
    return k(labpad, flat)


def _tc_body(x_ref, dot_ref, wsum_ref, out_ref):
    x = x_ref[0]
    dot = dot_ref[0]
    ws = wsum_ref[0]
    m = jnp.max(x, axis=1, keepdims=True)
    lse = m + jnp.log(jnp.sum(jnp.exp(x - m), axis=1, keepdims=True))
    out_ref[0, 0, 0] = jnp.sum(ws * lse - dot)


@jax.jit
def kernel(logits, label_ids):
    dot, wsum = _sc_sparse_part(logits, label_ids)
    parts = pl.pallas_call(
        _tc_body,
        grid=(NBLK,),
        in_specs=[
            pl.BlockSpec((1, ROWS, L), lambda i: (i, 0, 0)),
            pl.BlockSpec((1, ROWS, 1), lambda i: (i, 0, 0)),
            pl.BlockSpec((1, ROWS, 1), lambda i: (i, 0, 0)),
        ],
        out_specs=pl.BlockSpec((1, 1, 1), lambda i: (i, 0, 0),
                               memory_space=pltpu.SMEM),
        out_shape=jax.ShapeDtypeStruct((NBLK, 1, 1), jnp.float32),
        compiler_params=pltpu.CompilerParams(
            dimension_semantics=("parallel",)),
    )(logits.reshape(NBLK, ROWS, L),
      dot.reshape(NBLK, ROWS, 1),
      wsum.reshape(NBLK, ROWS, 1))
    return (jnp.sum(parts) / N).astype(jnp.float32)


# speedup vs baseline: 118.4658x; 1.6567x over previous
import jax
import jax.numpy as jnp
from jax import lax
from jax.experimental import pallas as pl
from jax.experimental.pallas import tpu as pltpu
from jax.experimental.pallas import tpu_sc as plsc

E = 0.1
CENTER = 1.0 - E
B, S, L = 16, 2048, 512
N = B * S
NC, NS = 2, 16
NW = NC * NS
TPW = N // NW
HALF = TPW // 2
ROWS = 512
NBLK = N // ROWS


def _sc_body(labpad_hbm, logits_hbm, dot_hbm, wsum_hbm,
             labs_v, xb_v, dot_v, wsum_v):
    cid = lax.axis_index("c")
    sid = lax.axis_index("s")
    wid = sid * NC + cid
    base = wid * TPW
    pltpu.sync_copy(labpad_hbm.at[pl.ds(base, TPW + 16)], labs_v)
    cbase = (wid % (S // TPW)) * TPW

    for h in range(TPW // HALF):
        pltpu.sync_copy(
            logits_hbm.at[pl.ds(base + h * HALF, HALF), pl.ds(0, 128)], xb_v)

        def group(g, carry):
            t0 = h * HALF + g * 16
            lane = lax.iota(jnp.int32, 16)
            tloc = t0 + lane
            c = cbase + tloc
            rs = []
            for j in range(-2, 3):
                rj = labs_v[pl.ds(t0 + 8 + j, 16)]
                vj = ((c + j) >= 0) & ((c + j) < S)
                rs.append(jnp.where(vj, rj, -1))
            dot = jnp.zeros(16, jnp.float32)
            ws = jnp.zeros(16, jnp.float32)
            for j in range(-2, 3):
                rj = rs[j + 2]
                bnd = (rj >= 0) & (rj < 16) & ((rj & 1) == 1)
                keep = bnd
                for jp in range(j + 1, 3):
                    keep = keep & (rs[jp + 2] != rj)
                if j == 0:
                    val = jnp.full(16, CENTER, jnp.float32)
                else:
                    cp = c + j
                    val = jnp.full(16, E / 4, jnp.float32)
                    val = jnp.where((cp == 1) | (cp == S - 2), E / 3, val)
                    val = jnp.where((cp == 0) | (cp == S - 1), E / 2, val)
                w = jnp.where(keep, val, 0.0)
                gj = plsc.load_gather(
                    xb_v, [g * 16 + lane, jnp.where(keep, rj, 0)])
                dot = dot + w * gj
                ws = ws + w
            r0 = rs[2]
            bnd0 = (r0 < 16) & ((r0 & 1) == 1)
            ws = ws + jnp.where(bnd0, 0.0, 1.0)
            dot_v[pl.ds(t0, 16)] = dot
            wsum_v[pl.ds(t0, 16)] = ws
            return carry

        lax.fori_loop(0, HALF // 16, group, 0)

    pltpu.sync_copy(dot_v, dot_hbm.at[pl.ds(base, TPW)])
    pltpu.sync_copy(wsum_v, wsum_hbm.at[pl.ds(base, TPW)])


def _sc_sparse_part(logits, label_ids):
    labpad = jnp.pad(label_ids, (8, 8), constant_values=-1)
    mesh = plsc.VectorSubcoreMesh(core_axis_name="c", subcore_axis_name="s",
                                  num_cores=NC, num_subcores=NS)
    k = pl.kernel(
        _sc_body,
        out_type=(jax.ShapeDtypeStruct((N,), jnp.float32),
                  jax.ShapeDtypeStruct((N,), jnp.float32)),
        mesh=mesh,
        scratch_types=[
            pltpu.VMEM((TPW + 16,), jnp.int32),
            pltpu.VMEM((HALF, 128), jnp.float32),
            pltpu.VMEM((TPW,), jnp.float32),
            pltpu.VMEM((TPW,), jnp.float32),
        ],
        compiler_params=pltpu.CompilerParams(needs_layout_passes=False),
    )
    return k(labpad, logits)


def _tc_body(x_ref, dot_ref, wsum_ref, lab_ref, out_ref):
    x = x_ref[0]
    r0 = lab_ref[...][:, None]
    bnd0 = (r0 < 16) & (r0 % 2 == 1)
    iota_l = jax.lax.broadcasted_iota(jnp.int32, (ROWS, L), 1)
    g0 = jnp.sum(jnp.where(iota_l == r0, x, 0.0), axis=1, keepdims=True)
    m = jnp.max(x, axis=1, keepdims=True)
    lse = m + jnp.log(jnp.sum(jnp.exp(x - m), axis=1, keepdims=True))
    ws = wsum_ref[...][:, None]
    dot = dot_ref[...][:, None] + jnp.where(bnd0, 0.0, g0)
    out_ref[0, 0, 0] = jnp.sum(ws * lse - dot)


@jax.jit
def kernel(logits, label_ids):
    dot, wsum = _sc_sparse_part(logits, label_ids)
    parts = pl.pallas_call(
        _tc_body,
        grid=(NBLK,),
        in_specs=[
            pl.BlockSpec((1, ROWS, L), lambda i: (i, 0, 0)),
            pl.BlockSpec((ROWS,), lambda i: (i,)),
            pl.BlockSpec((ROWS,), lambda i: (i,)),
            pl.BlockSpec((ROWS,), lambda i: (i,)),
        ],
        out_specs=pl.BlockSpec((1, 1, 1), lambda i: (i, 0, 0),
                               memory_space=pltpu.SMEM),
        out_shape=jax.ShapeDtypeStruct((NBLK, 1, 1), jnp.float32),
        compiler_params=pltpu.CompilerParams(
            dimension_semantics=("parallel",)),
    )(logits.reshape(NBLK, ROWS, L), dot, wsum, label_ids)
    return (jnp.sum(parts) / N).astype(jnp.float32)


# speedup vs baseline: 142.9245x; 1.2065x over previous
import jax
import jax.numpy as jnp
from jax import lax
from jax.experimental import pallas as pl
from jax.experimental.pallas import tpu as pltpu
from jax.experimental.pallas import tpu_sc as plsc

E = 0.1
CENTER = 1.0 - E
B, S, L = 16, 2048, 512
N = B * S
NC, NS = 2, 16
NW = NC * NS
TPW = N // NW
HALF = TPW // 2
ROWS = 512
NBLK = N // ROWS


def _sc_body(labpad_hbm, logits_hbm, dot_hbm, wsum_hbm,
             labs_v, xb_v, dot_v, wsum_v):
    cid = lax.axis_index("c")
    sid = lax.axis_index("s")
    wid = sid * NC + cid
    base = wid * TPW
    pltpu.sync_copy(labpad_hbm.at[pl.ds(base, TPW + 16)], labs_v)
    cbase = (wid % (S // TPW)) * TPW

    for h in range(TPW // HALF):
        pltpu.sync_copy(
            logits_hbm.at[pl.ds(base + h * HALF, HALF), pl.ds(0, 128)], xb_v)

        def group(g, carry):
            t0 = h * HALF + g * 16
            lane = lax.iota(jnp.int32, 16)
            tloc = t0 + lane
            c = cbase + tloc
            rs = []
            for j in range(-2, 3):
                rj = labs_v[pl.ds(t0 + 8 + j, 16)]
                vj = ((c + j) >= 0) & ((c + j) < S)
                rs.append(jnp.where(vj, rj, -1))
            dot = jnp.zeros(16, jnp.float32)
            ws = jnp.zeros(16, jnp.float32)
            for j in range(-2, 3):
                rj = rs[j + 2]
                bnd = (rj >= 0) & (rj < 16) & ((rj & 1) == 1)
                keep = bnd
                for jp in range(j + 1, 3):
                    keep = keep & (rs[jp + 2] != rj)
                if j == 0:
                    val = jnp.full(16, CENTER, jnp.float32)
                else:
                    cp = c + j
                    val = jnp.full(16, E / 4, jnp.float32)
                    val = jnp.where((cp == 1) | (cp == S - 2), E / 3, val)
                    val = jnp.where((cp == 0) | (cp == S - 1), E / 2, val)
                w = jnp.where(keep, val, 0.0)
                gj = plsc.load_gather(
                    xb_v, [g * 16 + lane, jnp.where(keep, rj, 0)])
                dot = dot + w * gj
                ws = ws + w
            r0 = rs[2]
            bnd0 = (r0 < 16) & ((r0 & 1) == 1)
            ws = ws + jnp.where(bnd0, 0.0, 1.0)
            dot_v[pl.ds(t0, 16)] = dot
            wsum_v[pl.ds(t0, 16)] = ws
            return carry

        lax.fori_loop(0, HALF // 16, group, 0)

    pltpu.sync_copy(dot_v, dot_hbm.at[pl.ds(base, TPW)])
    pltpu.sync_copy(wsum_v, wsum_hbm.at[pl.ds(base, TPW)])


def _sc_sparse_part(logits, label_ids):
    labpad = jnp.pad(label_ids, (8, 8), constant_values=-1)
    mesh = plsc.VectorSubcoreMesh(core_axis_name="c", subcore_axis_name="s",
                                  num_cores=NC, num_subcores=NS)
    k = pl.kernel(
        _sc_body,
        out_type=(jax.ShapeDtypeStruct((N,), jnp.float32),
                  jax.ShapeDtypeStruct((N,), jnp.float32)),
        mesh=mesh,
        scratch_types=[
            pltpu.VMEM((TPW + 16,), jnp.int32),
            pltpu.VMEM((HALF, 128), jnp.float32),
            pltpu.VMEM((TPW,), jnp.float32),
            pltpu.VMEM((TPW,), jnp.float32),
        ],
        compiler_params=pltpu.CompilerParams(needs_layout_passes=False),
    )
    return k(labpad, logits)


def _tc_lse_body(x_ref, lab_ref, lse_ref, own_ref):
    x = x_ref[0]
    r0 = lab_ref[...][:, None]
    bnd0 = (r0 < 16) & (r0 % 2 == 1)
    iota_l = jax.lax.broadcasted_iota(jnp.int32, (ROWS, L), 1)
    g0 = jnp.sum(jnp.where(iota_l == r0, x, 0.0), axis=1, keepdims=True)
    m = jnp.max(x, axis=1, keepdims=True)
    lse = m + jnp.log(jnp.sum(jnp.exp(x - m), axis=1, keepdims=True))
    lse_ref[...] = lse[:, 0]
    own_ref[0, 0, 0] = jnp.sum(jnp.where(bnd0, 0.0, g0))


def _tc_combine_body(lse_ref, dot_ref, wsum_ref, out_ref):
    out_ref[0, 0, 0] = jnp.sum(wsum_ref[...] * lse_ref[...] - dot_ref[...])


@jax.jit
def kernel(logits, label_ids):
    dot, wsum = _sc_sparse_part(logits, label_ids)
    lse, ownparts = pl.pallas_call(
        _tc_lse_body,
        grid=(NBLK,),
        in_specs=[
            pl.BlockSpec((1, ROWS, L), lambda i: (i, 0, 0)),
            pl.BlockSpec((ROWS,), lambda i: (i,)),
        ],
        out_specs=[
            pl.BlockSpec((ROWS,), lambda i: (i,)),
            pl.BlockSpec((1, 1, 1), lambda i: (i, 0, 0),
                         memory_space=pltpu.SMEM),
        ],
        out_shape=[
            jax.ShapeDtypeStruct((N,), jnp.float32),
            jax.ShapeDtypeStruct((NBLK, 1, 1), jnp.float32),
        ],
    )(logits.reshape(NBLK, ROWS, L), label_ids)
    combined = pl.pallas_call(
        _tc_combine_body,
        grid=(1,),
        in_specs=[
            pl.BlockSpec((N,), lambda i: (0,)),
            pl.BlockSpec((N,), lambda i: (0,)),
            pl.BlockSpec((N,), lambda i: (0,)),
        ],
        out_specs=pl.BlockSpec((1, 1, 1), lambda i: (0, 0, 0),
                               memory_space=pltpu.SMEM),
        out_shape=jax.ShapeDtypeStruct((1, 1, 1), jnp.float32),
    )(lse, dot, wsum)
    return ((combined[0, 0, 0] - jnp.sum(ownparts)) / N).astype(jnp.float32)


# speedup vs baseline: 182.9047x; 1.2797x over previous
import jax
import jax.numpy as jnp
from jax import lax
from jax.experimental import pallas as pl
from jax.experimental.pallas import tpu as pltpu
from jax.experimental.pallas import tpu_sc as plsc

E = 0.1
CENTER = 1.0 - E
B, S, L = 16, 2048, 512
N = B * S
NC, NS = 2, 16
NW = NC * NS
TPW = N // NW
HALF = TPW // 2
ROWS = 1024
NBLK = N // ROWS


def _sc_body(labels_hbm, logits_hbm, dot_hbm, wsum_hbm,
             labs_v, xb_v, dot_v, wsum_v):
    cid = lax.axis_index("c")
    sid = lax.axis_index("s")
    wid = sid * NC + cid
    base = wid * TPW
    @pl.when(wid == 0)
    def _():
        pltpu.sync_copy(labels_hbm.at[pl.ds(0, TPW + 8)],
                        labs_v.at[pl.ds(8, TPW + 8)])

    @pl.when(wid == NW - 1)
    def _():
        pltpu.sync_copy(labels_hbm.at[pl.ds(N - TPW - 8, TPW + 8)],
                        labs_v.at[pl.ds(0, TPW + 8)])

    @pl.when((wid > 0) & (wid < NW - 1))
    def _():
        pltpu.sync_copy(labels_hbm.at[pl.ds(base - 8, TPW + 16)], labs_v)

    cbase = (wid % (S // TPW)) * TPW

    for h in range(TPW // HALF):
        pltpu.sync_copy(
            logits_hbm.at[pl.ds(base + h * HALF, HALF), pl.ds(0, 128)], xb_v)

        def group(g, carry):
            t0 = h * HALF + g * 16
            lane = lax.iota(jnp.int32, 16)
            tloc = t0 + lane
            c = cbase + tloc
            rs = []
            for j in range(-2, 3):
                rj = labs_v[pl.ds(t0 + 8 + j, 16)]
                vj = ((c + j) >= 0) & ((c + j) < S)
                rs.append(jnp.where(vj, rj, -1))
            dot = jnp.zeros(16, jnp.float32)
            ws = jnp.zeros(16, jnp.float32)
            for j in range(-2, 3):
                rj = rs[j + 2]
                bnd = (rj >= 0) & (rj < 16) & ((rj & 1) == 1)
                keep = bnd
                for jp in range(j + 1, 3):
                    keep = keep & (rs[jp + 2] != rj)
                if j == 0:
                    val = jnp.full(16, CENTER, jnp.float32)
                else:
                    cp = c + j
                    val = jnp.full(16, E / 4, jnp.float32)
                    val = jnp.where((cp == 1) | (cp == S - 2), E / 3, val)
                    val = jnp.where((cp == 0) | (cp == S - 1), E / 2, val)
                w = jnp.where(keep, val, 0.0)
                gj = plsc.load_gather(
                    xb_v, [g * 16 + lane, jnp.where(keep, rj, 0)])
                dot = dot + w * gj
                ws = ws + w
            r0 = rs[2]
            bnd0 = (r0 < 16) & ((r0 & 1) == 1)
            ws = ws + jnp.where(bnd0, 0.0, 1.0)
            dot_v[pl.ds(t0, 16)] = dot
            wsum_v[pl.ds(t0, 16)] = ws
            return carry

        lax.fori_loop(0, HALF // 16, group, 0)

    pltpu.sync_copy(dot_v, dot_hbm.at[pl.ds(base, TPW)])
    pltpu.sync_copy(wsum_v, wsum_hbm.at[pl.ds(base, TPW)])


def _sc_sparse_part(logits, label_ids):
    mesh = plsc.VectorSubcoreMesh(core_axis_name="c", subcore_axis_name="s",
                                  num_cores=NC, num_subcores=NS)
    k = pl.kernel(
        _sc_body,
        out_type=(jax.ShapeDtypeStruct((N,), jnp.float32),
                  jax.ShapeDtypeStruct((N,), jnp.float32)),
        mesh=mesh,
        scratch_types=[
            pltpu.VMEM((TPW + 16,), jnp.int32),
            pltpu.VMEM((HALF, 128), jnp.float32),
            pltpu.VMEM((TPW,), jnp.float32),
            pltpu.VMEM((TPW,), jnp.float32),
        ],
        compiler_params=pltpu.CompilerParams(needs_layout_passes=False),
    )
    return k(label_ids, logits)


def _tc_lse_body(x_ref, lab_ref, lse_ref, own_ref):
    i = pl.program_id(0)
    x = x_ref[0]
    r0 = lab_ref[...][:, None]
    bnd0 = (r0 < 16) & (r0 % 2 == 1)
    iota_l = jax.lax.broadcasted_iota(jnp.int32, (ROWS, L), 1)
    g0 = jnp.sum(jnp.where(iota_l == r0, x, 0.0), axis=1, keepdims=True)
    m = jnp.max(x, axis=1, keepdims=True)
    lse = m + jnp.log(jnp.sum(jnp.exp(x - m), axis=1, keepdims=True))
    lse_ref[...] = lse[:, 0]

    @pl.when(i == 0)
    def _():
        own_ref[0, 0, 0] = 0.0
    own_ref[0, 0, 0] += jnp.sum(jnp.where(bnd0, 0.0, g0))


def _tc_combine_body(lse_ref, dot_ref, wsum_ref, own_ref, out_ref):
    tot = jnp.sum(wsum_ref[...] * lse_ref[...] - dot_ref[...])
    out_ref[0, 0, 0] = (tot - own_ref[0, 0, 0]) / N


@jax.jit
def kernel(logits, label_ids):
    dot, wsum = _sc_sparse_part(logits, label_ids)
    lse, own = pl.pallas_call(
        _tc_lse_body,
        grid=(NBLK,),
        in_specs=[
            pl.BlockSpec((1, ROWS, L), lambda i: (i, 0, 0)),
            pl.BlockSpec((ROWS,), lambda i: (i,)),
        ],
        out_specs=[
            pl.BlockSpec((ROWS,), lambda i: (i,)),
            pl.BlockSpec((1, 1, 1), lambda i: (0, 0, 0),
                         memory_space=pltpu.SMEM),
        ],
        out_shape=[
            jax.ShapeDtypeStruct((N,), jnp.float32),
            jax.ShapeDtypeStruct((1, 1, 1), jnp.float32),
        ],
    )(logits.reshape(NBLK, ROWS, L), label_ids)
    combined = pl.pallas_call(
        _tc_combine_body,
        grid=(1,),
        in_specs=[
            pl.BlockSpec((N,), lambda i: (0,)),
            pl.BlockSpec((N,), lambda i: (0,)),
            pl.BlockSpec((N,), lambda i: (0,)),
            pl.BlockSpec(memory_space=pltpu.SMEM),
        ],
        out_specs=pl.BlockSpec((1, 1, 1), lambda i: (0, 0, 0),
                               memory_space=pltpu.SMEM),
        out_shape=jax.ShapeDtypeStruct((1, 1, 1), jnp.float32),
    )(lse, dot, wsum, own)
    return combined[0, 0, 0]


# speedup vs baseline: 195.3656x; 1.0681x over previous
import jax
import jax.numpy as jnp
from jax import lax
from jax.experimental import pallas as pl
from jax.experimental.pallas import tpu as pltpu
from jax.experimental.pallas import tpu_sc as plsc

E = 0.1
CENTER = 1.0 - E
B, S, L = 16, 2048, 512
N = B * S
NC, NS = 2, 16
NW = NC * NS
TPW = N // NW
HALF = TPW // 2
ROWS = 2048
NBLK = N // ROWS


def _sc_body(labels_hbm, logits_hbm, dot_hbm, wsum_hbm,
             labs_v, xb_v, dot_v, wsum_v):
    cid = lax.axis_index("c")
    sid = lax.axis_index("s")
    wid = sid * NC + cid
    base = wid * TPW
    @pl.when(wid == 0)
    def _():
        pltpu.sync_copy(labels_hbm.at[pl.ds(0, TPW + 8)],
                        labs_v.at[pl.ds(8, TPW + 8)])

    @pl.when(wid == NW - 1)
    def _():
        pltpu.sync_copy(labels_hbm.at[pl.ds(N - TPW - 8, TPW + 8)],
                        labs_v.at[pl.ds(0, TPW + 8)])

    @pl.when((wid > 0) & (wid < NW - 1))
    def _():
        pltpu.sync_copy(labels_hbm.at[pl.ds(base - 8, TPW + 16)], labs_v)

    cbase = (wid % (S // TPW)) * TPW

    for h in range(TPW // HALF):
        pltpu.sync_copy(
            logits_hbm.at[pl.ds(base + h * HALF, HALF), pl.ds(0, 128)], xb_v)

        def group(g, carry):
            t0 = h * HALF + g * 16
            lane = lax.iota(jnp.int32, 16)
            tloc = t0 + lane
            c = cbase + tloc
            rs = []
            for j in range(-2, 3):
                rj = labs_v[pl.ds(t0 + 8 + j, 16)]
                vj = ((c + j) >= 0) & ((c + j) < S)
                rs.append(jnp.where(vj, rj, -1))
            dot = jnp.zeros(16, jnp.float32)
            ws = jnp.zeros(16, jnp.float32)
            for j in range(-2, 3):
                rj = rs[j + 2]
                bnd = (rj >= 0) & (rj < 16) & ((rj & 1) == 1)
                keep = bnd
                for jp in range(j + 1, 3):
                    keep = keep & (rs[jp + 2] != rj)
                if j == 0:
                    val = jnp.full(16, CENTER, jnp.float32)
                else:
                    cp = c + j
                    val = jnp.full(16, E / 4, jnp.float32)
                    val = jnp.where((cp == 1) | (cp == S - 2), E / 3, val)
                    val = jnp.where((cp == 0) | (cp == S - 1), E / 2, val)
                w = jnp.where(keep, val, 0.0)
                gj = plsc.load_gather(
                    xb_v, [g * 16 + lane, jnp.where(keep, rj, 0)])
                dot = dot + w * gj
                ws = ws + w
            r0 = rs[2]
            bnd0 = (r0 < 16) & ((r0 & 1) == 1)
            ws = ws + jnp.where(bnd0, 0.0, 1.0)
            dot_v[pl.ds(t0, 16)] = dot
            wsum_v[pl.ds(t0, 16)] = ws
            return carry

        lax.fori_loop(0, HALF // 16, group, 0)

    pltpu.sync_copy(dot_v, dot_hbm.at[pl.ds(base, TPW)])
    pltpu.sync_copy(wsum_v, wsum_hbm.at[pl.ds(base, TPW)])


def _sc_sparse_part(logits, label_ids):
    mesh = plsc.VectorSubcoreMesh(core_axis_name="c", subcore_axis_name="s",
                                  num_cores=NC, num_subcores=NS)
    k = pl.kernel(
        _sc_body,
        out_type=(jax.ShapeDtypeStruct((N,), jnp.float32),
                  jax.ShapeDtypeStruct((N,), jnp.float32)),
        mesh=mesh,
        scratch_types=[
            pltpu.VMEM((TPW + 16,), jnp.int32),
            pltpu.VMEM((HALF, 128), jnp.float32),
            pltpu.VMEM((TPW,), jnp.float32),
            pltpu.VMEM((TPW,), jnp.float32),
        ],
        compiler_params=pltpu.CompilerParams(needs_layout_passes=False),
    )
    return k(label_ids, logits)


def _tc_lse_body(x_ref, lab_ref, lse_ref, own_ref):
    i = pl.program_id(0)
    x = x_ref[0]
    r0 = lab_ref[...][:, None]
    bnd0 = (r0 < 16) & (r0 % 2 == 1)
    iota_l = jax.lax.broadcasted_iota(jnp.int32, (ROWS, L), 1)
    g0 = jnp.sum(jnp.where(iota_l == r0, x, 0.0), axis=1, keepdims=True)
    m = jnp.max(x, axis=1, keepdims=True)
    lse = m + jnp.log(jnp.sum(jnp.exp(x - m), axis=1, keepdims=True))
    lse_ref[...] = lse[:, 0]

    @pl.when(i == 0)
    def _():
        own_ref[0, 0, 0] = 0.0
    own_ref[0, 0, 0] += jnp.sum(jnp.where(bnd0, 0.0, g0))


def _tc_combine_body(lse_ref, dot_ref, wsum_ref, own_ref, out_ref):
    tot = jnp.sum(wsum_ref[...] * lse_ref[...] - dot_ref[...])
    out_ref[0, 0, 0] = (tot - own_ref[0, 0, 0]) / N


@jax.jit
def kernel(logits, label_ids):
    dot, wsum = _sc_sparse_part(logits, label_ids)
    lse, own = pl.pallas_call(
        _tc_lse_body,
        grid=(NBLK,),
        in_specs=[
            pl.BlockSpec((1, ROWS, L), lambda i: (i, 0, 0)),
            pl.BlockSpec((ROWS,), lambda i: (i,)),
        ],
        out_specs=[
            pl.BlockSpec((ROWS,), lambda i: (i,)),
            pl.BlockSpec((1, 1, 1), lambda i: (0, 0, 0),
                         memory_space=pltpu.SMEM),
        ],
        out_shape=[
            jax.ShapeDtypeStruct((N,), jnp.float32),
            jax.ShapeDtypeStruct((1, 1, 1), jnp.float32),
        ],
    )(logits.reshape(NBLK, ROWS, L), label_ids)
    combined = pl.pallas_call(
        _tc_combine_body,
        grid=(1,),
        in_specs=[
            pl.BlockSpec((N,), lambda i: (0,)),
            pl.BlockSpec((N,), lambda i: (0,)),
            pl.BlockSpec((N,), lambda i: (0,)),
            pl.BlockSpec(memory_space=pltpu.SMEM),
        ],
        out_specs=pl.BlockSpec((1, 1, 1), lambda i: (0, 0, 0),
                               memory_space=pltpu.SMEM),
        out_shape=jax.ShapeDtypeStruct((1, 1, 1), jnp.float32),
    )(lse, dot, wsum, own)
    return combined[0, 0, 0]


# speedup vs baseline: 200.9534x; 1.0286x over previous
import jax
import jax.numpy as jnp
from jax import lax
from jax.experimental import pallas as pl
from jax.experimental.pallas import tpu as pltpu
from jax.experimental.pallas import tpu_sc as plsc

E = 0.1
CENTER = 1.0 - E
B, S, L = 16, 2048, 512
N = B * S
NC, NS = 2, 16
NW = NC * NS
TPW = N // NW
HALF = TPW // 2
ROWS = 4096
NBLK = N // ROWS


def _sc_body(labels_hbm, logits_hbm, dot_hbm, wsum_hbm,
             labs_v, xb_v, dot_v, wsum_v):
    cid = lax.axis_index("c")
    sid = lax.axis_index("s")
    wid = sid * NC + cid
    base = wid * TPW
    @pl.when(wid == 0)
    def _():
        pltpu.sync_copy(labels_hbm.at[pl.ds(0, TPW + 8)],
                        labs_v.at[pl.ds(8, TPW + 8)])

    @pl.when(wid == NW - 1)
    def _():
        pltpu.sync_copy(labels_hbm.at[pl.ds(N - TPW - 8, TPW + 8)],
                        labs_v.at[pl.ds(0, TPW + 8)])

    @pl.when((wid > 0) & (wid < NW - 1))
    def _():
        pltpu.sync_copy(labels_hbm.at[pl.ds(base - 8, TPW + 16)], labs_v)

    cbase = (wid % (S // TPW)) * TPW

    for h in range(TPW // HALF):
        pltpu.sync_copy(
            logits_hbm.at[pl.ds(base + h * HALF, HALF), pl.ds(0, 128)], xb_v)

        def group(g, carry):
            t0 = h * HALF + g * 16
            lane = lax.iota(jnp.int32, 16)
            tloc = t0 + lane
            c = cbase + tloc
            rs = []
            for j in range(-2, 3):
                rj = labs_v[pl.ds(t0 + 8 + j, 16)]
                vj = ((c + j) >= 0) & ((c + j) < S)
                rs.append(jnp.where(vj, rj, -1))
            dot = jnp.zeros(16, jnp.float32)
            ws = jnp.zeros(16, jnp.float32)
            for j in range(-2, 3):
                rj = rs[j + 2]
                bnd = (rj >= 0) & (rj < 16) & ((rj & 1) == 1)
                keep = bnd
                for jp in range(j + 1, 3):
                    keep = keep & (rs[jp + 2] != rj)
                if j == 0:
                    val = jnp.full(16, CENTER, jnp.float32)
                else:
                    cp = c + j
                    val = jnp.full(16, E / 4, jnp.float32)
                    val = jnp.where((cp == 1) | (cp == S - 2), E / 3, val)
                    val = jnp.where((cp == 0) | (cp == S - 1), E / 2, val)
                w = jnp.where(keep, val, 0.0)
                gj = plsc.load_gather(
                    xb_v, [g * 16 + lane, jnp.where(keep, rj, 0)])
                dot = dot + w * gj
                ws = ws + w
            r0 = rs[2]
            bnd0 = (r0 < 16) & ((r0 & 1) == 1)
            ws = ws + jnp.where(bnd0, 0.0, 1.0)
            dot_v[pl.ds(t0, 16)] = dot
            wsum_v[pl.ds(t0, 16)] = ws
            return carry

        lax.fori_loop(0, HALF // 16, group, 0)

    pltpu.sync_copy(dot_v, dot_hbm.at[pl.ds(base, TPW)])
    pltpu.sync_copy(wsum_v, wsum_hbm.at[pl.ds(base, TPW)])


def _sc_sparse_part(logits, label_ids):
    mesh = plsc.VectorSubcoreMesh(core_axis_name="c", subcore_axis_name="s",
                                  num_cores=NC, num_subcores=NS)
    k = pl.kernel(
        _sc_body,
        out_type=(jax.ShapeDtypeStruct((N,), jnp.float32),
                  jax.ShapeDtypeStruct((N,), jnp.float32)),
        mesh=mesh,
        scratch_types=[
            pltpu.VMEM((TPW + 16,), jnp.int32),
            pltpu.VMEM((HALF, 128), jnp.float32),
            pltpu.VMEM((TPW,), jnp.float32),
            pltpu.VMEM((TPW,), jnp.float32),
        ],
        compiler_params=pltpu.CompilerParams(needs_layout_passes=False),
    )
    return k(label_ids, logits)


def _tc_lse_body(x_ref, lab_ref, lse_ref, own_ref):
    i = pl.program_id(0)
    x = x_ref[0]
    r0 = lab_ref[...][:, None]
    bnd0 = (r0 < 16) & (r0 % 2 == 1)
    iota_l = jax.lax.broadcasted_iota(jnp.int32, (ROWS, L), 1)
    g0 = jnp.sum(jnp.where(iota_l == r0, x, 0.0), axis=1, keepdims=True)
    m = jnp.max(x, axis=1, keepdims=True)
    lse = m + jnp.log(jnp.sum(jnp.exp(x - m), axis=1, keepdims=True))
    lse_ref[...] = lse[:, 0]

    @pl.when(i == 0)
    def _():
        own_ref[0, 0, 0] = 0.0
    own_ref[0, 0, 0] += jnp.sum(jnp.where(bnd0, 0.0, g0))


def _tc_combine_body(lse_ref, dot_ref, wsum_ref, own_ref, out_ref):
    tot = jnp.sum(wsum_ref[...] * lse_ref[...] - dot_ref[...])
    out_ref[0, 0, 0] = (tot - own_ref[0, 0, 0]) / N


@jax.jit
def kernel(logits, label_ids):
    dot, wsum = _sc_sparse_part(logits, label_ids)
    lse, own = pl.pallas_call(
        _tc_lse_body,
        grid=(NBLK,),
        in_specs=[
            pl.BlockSpec((1, ROWS, L), lambda i: (i, 0, 0)),
            pl.BlockSpec((ROWS,), lambda i: (i,)),
        ],
        out_specs=[
            pl.BlockSpec((ROWS,), lambda i: (i,)),
            pl.BlockSpec((1, 1, 1), lambda i: (0, 0, 0),
                         memory_space=pltpu.SMEM),
        ],
        out_shape=[
            jax.ShapeDtypeStruct((N,), jnp.float32),
            jax.ShapeDtypeStruct((1, 1, 1), jnp.float32),
        ],
    )(logits.reshape(NBLK, ROWS, L), label_ids)
    combined = pl.pallas_call(
        _tc_combine_body,
        grid=(1,),
        in_specs=[
            pl.BlockSpec((N,), lambda i: (0,)),
            pl.BlockSpec((N,), lambda i: (0,)),
            pl.BlockSpec((N,), lambda i: (0,)),
            pl.BlockSpec(memory_space=pltpu.SMEM),
        ],
        out_specs=pl.BlockSpec((1, 1, 1), lambda i: (0, 0, 0),
                               memory_space=pltpu.SMEM),
        out_shape=jax.ShapeDtypeStruct((1, 1, 1), jnp.float32),
    )(lse, dot, wsum, own)
    return combined[0, 0, 0]
